# Initial kernel scaffold; baseline (speedup 1.0000x reference)
import jax
import jax.numpy as jnp
from jax.experimental import pallas as pl


def kernel(tokens, embedding_weight):
    raise NotImplementedError("write your pallas kernel here")


# speedup vs baseline: 1.5617x; 1.5617x over previous
import functools

import jax
import jax.numpy as jnp
from jax import lax
from jax.experimental import pallas as pl
from jax.experimental.pallas import tpu as pltpu
from jax.experimental.pallas import tpu_sc as plsc

VOCAB = 100000
D_MODEL = 1024
BATCH = 4
SEQ_LEN = 2048

NUM_CORES = 2
NUM_SUBCORES = 16
NW = NUM_CORES * NUM_SUBCORES
B_TOTAL = BATCH * SEQ_LEN
B_PER_W = B_TOTAL // NW
CHUNK = 32
NCHUNK = B_PER_W // CHUNK
NBUF = 3


def _emb_body(tok_hbm, table_hbm, out_hbm, idx_v, b0, b1, b2, gsem, ssem):
    bufs = (b0, b1, b2)
    wid = lax.axis_index("s") * NUM_CORES + lax.axis_index("c")
    base = wid * B_PER_W
    pltpu.sync_copy(tok_hbm.at[wid], idx_v)

    gat = [None] * NCHUNK
    sca = [None] * NCHUNK
    for c in range(NBUF):
        gat[c] = pltpu.make_async_copy(
            table_hbm.at[idx_v.at[c]], bufs[c], gsem)
        gat[c].start()
    for c in range(NCHUNK):
        gat[c].wait()
        sca[c] = pltpu.make_async_copy(
            bufs[c % NBUF], out_hbm.at[pl.ds(base + c * CHUNK, CHUNK)], ssem)
        sca[c].start()
        nxt = c + NBUF
        if nxt < NCHUNK:
            sca[nxt - NBUF].wait()
            gat[nxt] = pltpu.make_async_copy(
                table_hbm.at[idx_v.at[nxt]], bufs[nxt % NBUF], gsem)
            gat[nxt].start()
    for c in range(NCHUNK - NBUF, NCHUNK):
        sca[c].wait()


@jax.jit
def _embedding_lookup(tokens_grouped, table):
    mesh = plsc.VectorSubcoreMesh(core_axis_name="c", subcore_axis_name="s")
    run = pl.kernel(
        _emb_body,
        out_type=jax.ShapeDtypeStruct((B_TOTAL, D_MODEL), jnp.float32),
        mesh=mesh,
        scratch_types=[
            pltpu.VMEM((NCHUNK, CHUNK), jnp.int32),
            pltpu.VMEM((CHUNK, D_MODEL), jnp.float32),
            pltpu.VMEM((CHUNK, D_MODEL), jnp.float32),
            pltpu.VMEM((CHUNK, D_MODEL), jnp.float32),
            pltpu.SemaphoreType.DMA,
            pltpu.SemaphoreType.DMA,
        ],
    )
    return run(tokens_grouped, table)


def kernel(tokens, embedding_weight):
    tok = tokens.astype(jnp.int32).reshape(NW, NCHUNK, CHUNK)
    out = _embedding_lookup(tok, embedding_weight)
    return out.reshape(BATCH, SEQ_LEN, D_MODEL)
